# Initial kernel scaffold; baseline (speedup 1.0000x reference)
import jax
import jax.numpy as jnp
from jax.experimental import pallas as pl


def kernel(P, gumbel, Tau, mask_rows, mask_cols):
    raise NotImplementedError("write your pallas kernel here")


# speedup vs baseline: 27.1584x; 27.1584x over previous
import jax
import jax.numpy as jnp
from jax import lax
from jax.experimental import pallas as pl
from jax.experimental.pallas import tpu as pltpu, tpu_sc as plsc

TWO_PIN_NET_NUM = 100000
L_PATTERN_NUM = 4
NPF = TWO_PIN_NET_NUM * L_PATTERN_NUM
G = 2 * 512 * 512
NNZ = 8000000
CAPACITY = 2.0
WCONG = 1.0

NC = 2
NS = 16
NW = NC * NS

PF_PER_SUB = 25008
NPF_PAD = NS * PF_PER_SUB
G_PER_SUB = G // NS
ZCH = 2048

ROWS = NNZ // 128
ROWS_BASE = ROWS // NW
ROWS_EXTRA = ROWS - ROWS_BASE * NW


_SHUF_DNUMS = lax.GatherDimensionNumbers(
    offset_dims=(), collapsed_slice_dims=(0,), start_index_map=(0,))


def _shuffle(x, idx2):
    return lax.gather(x, idx2, _SHUF_DNUMS, (1,),
                      mode=lax.GatherScatterMode.PROMISE_IN_BOUNDS)


def _sc_body(p1_hbm, g1_hbm, tau_hbm, rows_hbm, cols_hbm,
             d0_hbm, d1_hbm,
             pbuf, gbuf, taub, zbuf, colsv, rowsv, valsv,
             p_sh, dem_sh, sem):
    c = lax.axis_index("c")
    s = lax.axis_index("s")
    wid = s * NC + c

    base_pf = s * PF_PER_SUB
    pltpu.sync_copy(p1_hbm.at[pl.ds(base_pf, PF_PER_SUB)], pbuf)
    pltpu.sync_copy(g1_hbm.at[pl.ds(base_pf, PF_PER_SUB)], gbuf)
    pltpu.sync_copy(tau_hbm, taub)
    tau = taub[...]
    lane = lax.iota(jnp.int32, 16)
    perm1 = (lane ^ 1)[:, None]
    perm2 = (lane ^ 2)[:, None]

    def sm_body(j, carry):
        b = j * 16
        x = (pbuf[pl.ds(b, 16)] + gbuf[pl.ds(b, 16)]) / tau
        m = jnp.maximum(x, _shuffle(x, perm1))
        m = jnp.maximum(m, _shuffle(m, perm2))
        e = jnp.exp(x - m)
        t = e + _shuffle(e, perm1)
        t = t + _shuffle(t, perm2)
        pbuf[pl.ds(b, 16)] = e / t
        return carry

    lax.fori_loop(0, PF_PER_SUB // 16, sm_body, 0)
    pltpu.sync_copy(pbuf, p_sh.at[pl.ds(base_pf, PF_PER_SUB)])

    def z_body(k, carry):
        zbuf[pl.ds(k * 16, 16)] = jnp.zeros((16,), jnp.float32)
        return carry

    lax.fori_loop(0, ZCH // 16, z_body, 0)

    def zc_body(k, carry):
        pltpu.sync_copy(zbuf, dem_sh.at[pl.ds(s * G_PER_SUB + k * ZCH, ZCH)])
        return carry

    lax.fori_loop(0, G_PER_SUB // ZCH, zc_body, 0)

    plsc.subcore_barrier()

    start = wid * ROWS_BASE + jnp.minimum(wid, ROWS_EXTRA)
    n_rows = ROWS_BASE + jnp.where(wid < ROWS_EXTRA, 1, 0)

    def w_body(r, carry):
        row = start + r
        pltpu.sync_copy(cols_hbm.at[row], colsv)
        pltpu.sync_copy(rows_hbm.at[row], rowsv)
        pltpu.async_copy(p_sh.at[colsv], valsv, sem).wait()
        pltpu.sync_copy(valsv, dem_sh.at[rowsv], add=True)
        return carry

    lax.fori_loop(0, n_rows, w_body, 0)

    plsc.subcore_barrier()

    off = s * G_PER_SUB

    @pl.when(c == 0)
    def _():
        pltpu.sync_copy(dem_sh.at[pl.ds(off, G_PER_SUB)],
                        d0_hbm.at[pl.ds(off, G_PER_SUB)])

    @pl.when(c == 1)
    def _():
        pltpu.sync_copy(dem_sh.at[pl.ds(off, G_PER_SUB)],
                        d1_hbm.at[pl.ds(off, G_PER_SUB)])


_sc_call = pl.kernel(
    _sc_body,
    out_type=(jax.ShapeDtypeStruct((G,), jnp.float32),
              jax.ShapeDtypeStruct((G,), jnp.float32)),
    mesh=plsc.VectorSubcoreMesh(core_axis_name="c", subcore_axis_name="s"),
    scratch_types=[
        pltpu.VMEM((PF_PER_SUB,), jnp.float32),
        pltpu.VMEM((PF_PER_SUB,), jnp.float32),
        pltpu.VMEM((16,), jnp.float32),
        pltpu.VMEM((ZCH,), jnp.float32),
        pltpu.VMEM((128,), jnp.int32),
        pltpu.VMEM((128,), jnp.int32),
        pltpu.VMEM((128,), jnp.float32),
        pltpu.VMEM_SHARED((NPF_PAD,), jnp.float32),
        pltpu.VMEM_SHARED((G,), jnp.float32),
        pltpu.SemaphoreType.DMA,
    ],
)

RED_ROWS = G // 128
RED_BLK = 512
RED_GRID = RED_ROWS // RED_BLK


def _red_body(d0_ref, d1_ref, o_ref):
    i = pl.program_id(0)
    x = d0_ref[...] + d1_ref[...]
    part = jnp.sum(jnp.maximum(x - CAPACITY, 0.0)) * WCONG

    @pl.when(i == 0)
    def _():
        o_ref[...] = jnp.zeros_like(o_ref)

    row = lax.broadcasted_iota(jnp.int32, (8, 128), 0)
    col = lax.broadcasted_iota(jnp.int32, (8, 128), 1)
    o_ref[...] += jnp.where((row == 0) & (col == 0), part, 0.0)


_red_call = pl.pallas_call(
    _red_body,
    grid=(RED_GRID,),
    in_specs=[pl.BlockSpec((RED_BLK, 128), lambda i: (i, 0)),
              pl.BlockSpec((RED_BLK, 128), lambda i: (i, 0))],
    out_specs=pl.BlockSpec((8, 128), lambda i: (0, 0)),
    out_shape=jax.ShapeDtypeStruct((8, 128), jnp.float32),
)


def kernel(P, gumbel, Tau, mask_rows, mask_cols):
    p1 = jnp.pad(P.reshape(NPF), (0, NPF_PAD - NPF))
    g1 = jnp.pad(gumbel.reshape(NPF), (0, NPF_PAD - NPF))
    taub = jnp.full((16,), Tau, jnp.float32)
    rows2 = mask_rows.reshape(ROWS, 128)
    cols2 = mask_cols.reshape(ROWS, 128)
    d0, d1 = _sc_call(p1, g1, taub, rows2, cols2)
    acc = _red_call(d0.reshape(RED_ROWS, 128), d1.reshape(RED_ROWS, 128))
    return acc[0, 0]


# speedup vs baseline: 74.1577x; 2.7306x over previous
import jax
import jax.numpy as jnp
from jax import lax
from jax.experimental import pallas as pl
from jax.experimental.pallas import tpu as pltpu, tpu_sc as plsc

TWO_PIN_NET_NUM = 100000
L_PATTERN_NUM = 4
NPF = TWO_PIN_NET_NUM * L_PATTERN_NUM
G = 2 * 512 * 512
NNZ = 8000000
CAPACITY = 2.0
WCONG = 1.0

NC = 2
NS = 16
NW = NC * NS

PF_PER_SUB = 25008
NPF_PAD = NS * PF_PER_SUB
G_PER_SUB = G // NS
ZCH = 2048

ROWS = NNZ // 128
CHK = 16
ROWS_PAD = ((ROWS + NW * CHK - 1) // (NW * CHK)) * (NW * CHK)
RPW = ROWS_PAD // NW
N_CHUNKS = RPW // CHK
PADN = (ROWS_PAD - ROWS) * 128
PAD_LOCAL = NPF - (NS - 1) * PF_PER_SUB


_SHUF_DNUMS = lax.GatherDimensionNumbers(
    offset_dims=(), collapsed_slice_dims=(0,), start_index_map=(0,))


def _shuffle(x, idx2):
    return lax.gather(x, idx2, _SHUF_DNUMS, (1,),
                      mode=lax.GatherScatterMode.PROMISE_IN_BOUNDS)


def _sc_body(p1_hbm, g1_hbm, tau_hbm, rows_hbm, cols_hbm,
             d0_hbm, d1_hbm,
             pbuf, gbuf, taub, zbuf, colsb, rowsb, valsb,
             p_sh, dem_sh, gsem, ssem):
    c = lax.axis_index("c")
    s = lax.axis_index("s")
    wid = s * NC + c

    base_pf = s * PF_PER_SUB
    pltpu.sync_copy(p1_hbm.at[pl.ds(base_pf, PF_PER_SUB)], pbuf)
    pltpu.sync_copy(g1_hbm.at[pl.ds(base_pf, PF_PER_SUB)], gbuf)
    pltpu.sync_copy(tau_hbm, taub)
    tau = taub[...]
    lane = lax.iota(jnp.int32, 16)
    perm1 = (lane ^ 1)[:, None]
    perm2 = (lane ^ 2)[:, None]

    def sm_body(j, carry):
        b = j * 16
        x = (pbuf[pl.ds(b, 16)] + gbuf[pl.ds(b, 16)]) / tau
        m = jnp.maximum(x, _shuffle(x, perm1))
        m = jnp.maximum(m, _shuffle(m, perm2))
        e = jnp.exp(x - m)
        t = e + _shuffle(e, perm1)
        t = t + _shuffle(t, perm2)
        pbuf[pl.ds(b, 16)] = e / t
        return carry

    lax.fori_loop(0, PF_PER_SUB // 16, sm_body, 0)

    @pl.when(s == NS - 1)
    def _():
        def zp_body(k, carry):
            pbuf[pl.ds(PAD_LOCAL + k * 16, 16)] = jnp.zeros((16,), jnp.float32)
            return carry

        lax.fori_loop(0, (PF_PER_SUB - PAD_LOCAL) // 16, zp_body, 0)

    pltpu.sync_copy(pbuf, p_sh.at[pl.ds(base_pf, PF_PER_SUB)])

    def z_body(k, carry):
        zbuf[pl.ds(k * 16, 16)] = jnp.zeros((16,), jnp.float32)
        return carry

    lax.fori_loop(0, ZCH // 16, z_body, 0)

    def zc_body(k, carry):
        pltpu.sync_copy(zbuf, dem_sh.at[pl.ds(s * G_PER_SUB + k * ZCH, ZCH)])
        return carry

    lax.fori_loop(0, G_PER_SUB // ZCH, zc_body, 0)

    plsc.subcore_barrier()

    start = wid * RPW

    def c_body(ch, carry):
        rowbase = start + ch * CHK
        pltpu.sync_copy(cols_hbm.at[pl.ds(rowbase, CHK)], colsb)
        pltpu.sync_copy(rows_hbm.at[pl.ds(rowbase, CHK)], rowsb)
        for j in range(CHK):
            pltpu.async_copy(p_sh.at[colsb.at[j]], valsb.at[j],
                             gsem.at[j]).wait()
            pltpu.sync_copy(valsb.at[j], dem_sh.at[rowsb.at[j]], add=True)
        return carry

    lax.fori_loop(0, N_CHUNKS, c_body, 0)

    plsc.subcore_barrier()

    off = s * G_PER_SUB

    @pl.when(c == 0)
    def _():
        pltpu.sync_copy(dem_sh.at[pl.ds(off, G_PER_SUB)],
                        d0_hbm.at[pl.ds(off, G_PER_SUB)])

    @pl.when(c == 1)
    def _():
        pltpu.sync_copy(dem_sh.at[pl.ds(off, G_PER_SUB)],
                        d1_hbm.at[pl.ds(off, G_PER_SUB)])


_sc_call = pl.kernel(
    _sc_body,
    out_type=(jax.ShapeDtypeStruct((G,), jnp.float32),
              jax.ShapeDtypeStruct((G,), jnp.float32)),
    mesh=plsc.VectorSubcoreMesh(core_axis_name="c", subcore_axis_name="s"),
    scratch_types=[
        pltpu.VMEM((PF_PER_SUB,), jnp.float32),
        pltpu.VMEM((PF_PER_SUB,), jnp.float32),
        pltpu.VMEM((16,), jnp.float32),
        pltpu.VMEM((ZCH,), jnp.float32),
        pltpu.VMEM((CHK, 128), jnp.int32),
        pltpu.VMEM((CHK, 128), jnp.int32),
        pltpu.VMEM((CHK, 128), jnp.float32),
        pltpu.VMEM_SHARED((NPF_PAD,), jnp.float32),
        pltpu.VMEM_SHARED((G,), jnp.float32),
        pltpu.SemaphoreType.DMA((CHK,)),
        pltpu.SemaphoreType.DMA((CHK,)),
    ],
)

RED_ROWS = G // 128
RED_BLK = 512
RED_GRID = RED_ROWS // RED_BLK


def _red_body(d0_ref, d1_ref, o_ref):
    i = pl.program_id(0)
    x = d0_ref[...] + d1_ref[...]
    part = jnp.sum(jnp.maximum(x - CAPACITY, 0.0)) * WCONG

    @pl.when(i == 0)
    def _():
        o_ref[...] = jnp.zeros_like(o_ref)

    row = lax.broadcasted_iota(jnp.int32, (8, 128), 0)
    col = lax.broadcasted_iota(jnp.int32, (8, 128), 1)
    o_ref[...] += jnp.where((row == 0) & (col == 0), part, 0.0)


_red_call = pl.pallas_call(
    _red_body,
    grid=(RED_GRID,),
    in_specs=[pl.BlockSpec((RED_BLK, 128), lambda i: (i, 0)),
              pl.BlockSpec((RED_BLK, 128), lambda i: (i, 0))],
    out_specs=pl.BlockSpec((8, 128), lambda i: (0, 0)),
    out_shape=jax.ShapeDtypeStruct((8, 128), jnp.float32),
)


def kernel(P, gumbel, Tau, mask_rows, mask_cols):
    p1 = jnp.pad(P.reshape(NPF), (0, NPF_PAD - NPF))
    g1 = jnp.pad(gumbel.reshape(NPF), (0, NPF_PAD - NPF))
    taub = jnp.full((16,), Tau, jnp.float32)
    rows_p = jnp.concatenate(
        [mask_rows.astype(jnp.int32), jnp.zeros((PADN,), jnp.int32)])
    cols_p = jnp.concatenate(
        [mask_cols.astype(jnp.int32), jnp.full((PADN,), NPF, jnp.int32)])
    rows2 = rows_p.reshape(ROWS_PAD, 128)
    cols2 = cols_p.reshape(ROWS_PAD, 128)
    d0, d1 = _sc_call(p1, g1, taub, rows2, cols2)
    acc = _red_call(d0.reshape(RED_ROWS, 128), d1.reshape(RED_ROWS, 128))
    return acc[0, 0]


# speedup vs baseline: 101.7698x; 1.3723x over previous
import jax
import jax.numpy as jnp
from jax import lax
from jax.experimental import pallas as pl
from jax.experimental.pallas import tpu as pltpu, tpu_sc as plsc

TWO_PIN_NET_NUM = 100000
L_PATTERN_NUM = 4
NPF = TWO_PIN_NET_NUM * L_PATTERN_NUM
G = 2 * 512 * 512
NNZ = 8000000
CAPACITY = 2.0
WCONG = 1.0

NC = 2
NS = 16
NW = NC * NS

PF_PER_SUB = 25008
NPF_PAD = NS * PF_PER_SUB
G_PER_SUB = G // NS
ZCH = 2048

ROWS = NNZ // 128
CHK = 16
ROWS_PAD = ((ROWS + NW * CHK - 1) // (NW * CHK)) * (NW * CHK)
RPW = ROWS_PAD // NW
N_CHUNKS = RPW // CHK
PADN = (ROWS_PAD - ROWS) * 128
PAD_LOCAL = NPF - (NS - 1) * PF_PER_SUB


_SHUF_DNUMS = lax.GatherDimensionNumbers(
    offset_dims=(), collapsed_slice_dims=(0,), start_index_map=(0,))


def _shuffle(x, idx2):
    return lax.gather(x, idx2, _SHUF_DNUMS, (1,),
                      mode=lax.GatherScatterMode.PROMISE_IN_BOUNDS)


def _sc_body(p1_hbm, g1_hbm, tau_hbm, rows_hbm, cols_hbm,
             d0_hbm, d1_hbm,
             pbuf, gbuf, taub, zbuf, colsb, rowsb, valsb,
             p_sh, dem_sh):
    c = lax.axis_index("c")
    s = lax.axis_index("s")
    wid = s * NC + c

    base_pf = s * PF_PER_SUB
    pltpu.sync_copy(p1_hbm.at[pl.ds(base_pf, PF_PER_SUB)], pbuf)
    pltpu.sync_copy(g1_hbm.at[pl.ds(base_pf, PF_PER_SUB)], gbuf)
    pltpu.sync_copy(tau_hbm, taub)
    tau = taub[...]
    lane = lax.iota(jnp.int32, 16)
    perm1 = (lane ^ 1)[:, None]
    perm2 = (lane ^ 2)[:, None]

    def sm_body(j, carry):
        b = j * 16
        x = (pbuf[pl.ds(b, 16)] + gbuf[pl.ds(b, 16)]) / tau
        m = jnp.maximum(x, _shuffle(x, perm1))
        m = jnp.maximum(m, _shuffle(m, perm2))
        e = jnp.exp(x - m)
        t = e + _shuffle(e, perm1)
        t = t + _shuffle(t, perm2)
        pbuf[pl.ds(b, 16)] = e / t
        return carry

    lax.fori_loop(0, PF_PER_SUB // 16, sm_body, 0)

    @pl.when(s == NS - 1)
    def _():
        def zp_body(k, carry):
            pbuf[pl.ds(PAD_LOCAL + k * 16, 16)] = jnp.zeros((16,), jnp.float32)
            return carry

        lax.fori_loop(0, (PF_PER_SUB - PAD_LOCAL) // 16, zp_body, 0)

    pltpu.sync_copy(pbuf, p_sh.at[pl.ds(base_pf, PF_PER_SUB)])

    def z_body(k, carry):
        zbuf[pl.ds(k * 16, 16)] = jnp.zeros((16,), jnp.float32)
        return carry

    lax.fori_loop(0, ZCH // 16, z_body, 0)

    def zc_body(k, carry):
        pltpu.sync_copy(zbuf, dem_sh.at[pl.ds(s * G_PER_SUB + k * ZCH, ZCH)])
        return carry

    lax.fori_loop(0, G_PER_SUB // ZCH, zc_body, 0)

    plsc.subcore_barrier()

    start = wid * RPW

    def c_body(ch, carry):
        base = (start + ch * CHK) * 128
        pltpu.sync_copy(cols_hbm.at[pl.ds(base, CHK * 128)], colsb)
        pltpu.sync_copy(rows_hbm.at[pl.ds(base, CHK * 128)], rowsb)
        pltpu.sync_copy(p_sh.at[colsb], valsb)
        pltpu.sync_copy(valsb, dem_sh.at[rowsb], add=True)
        return carry

    lax.fori_loop(0, N_CHUNKS, c_body, 0)

    plsc.subcore_barrier()

    off = s * G_PER_SUB

    @pl.when(c == 0)
    def _():
        pltpu.sync_copy(dem_sh.at[pl.ds(off, G_PER_SUB)],
                        d0_hbm.at[pl.ds(off, G_PER_SUB)])

    @pl.when(c == 1)
    def _():
        pltpu.sync_copy(dem_sh.at[pl.ds(off, G_PER_SUB)],
                        d1_hbm.at[pl.ds(off, G_PER_SUB)])


_sc_call = pl.kernel(
    _sc_body,
    out_type=(jax.ShapeDtypeStruct((G,), jnp.float32),
              jax.ShapeDtypeStruct((G,), jnp.float32)),
    mesh=plsc.VectorSubcoreMesh(core_axis_name="c", subcore_axis_name="s"),
    scratch_types=[
        pltpu.VMEM((PF_PER_SUB,), jnp.float32),
        pltpu.VMEM((PF_PER_SUB,), jnp.float32),
        pltpu.VMEM((16,), jnp.float32),
        pltpu.VMEM((ZCH,), jnp.float32),
        pltpu.VMEM((CHK * 128,), jnp.int32),
        pltpu.VMEM((CHK * 128,), jnp.int32),
        pltpu.VMEM((CHK * 128,), jnp.float32),
        pltpu.VMEM_SHARED((NPF_PAD,), jnp.float32),
        pltpu.VMEM_SHARED((G,), jnp.float32),
    ],
)

RED_ROWS = G // 128
RED_BLK = 512
RED_GRID = RED_ROWS // RED_BLK


def _red_body(d0_ref, d1_ref, o_ref):
    i = pl.program_id(0)
    x = d0_ref[...] + d1_ref[...]
    part = jnp.sum(jnp.maximum(x - CAPACITY, 0.0)) * WCONG

    @pl.when(i == 0)
    def _():
        o_ref[...] = jnp.zeros_like(o_ref)

    row = lax.broadcasted_iota(jnp.int32, (8, 128), 0)
    col = lax.broadcasted_iota(jnp.int32, (8, 128), 1)
    o_ref[...] += jnp.where((row == 0) & (col == 0), part, 0.0)


_red_call = pl.pallas_call(
    _red_body,
    grid=(RED_GRID,),
    in_specs=[pl.BlockSpec((RED_BLK, 128), lambda i: (i, 0)),
              pl.BlockSpec((RED_BLK, 128), lambda i: (i, 0))],
    out_specs=pl.BlockSpec((8, 128), lambda i: (0, 0)),
    out_shape=jax.ShapeDtypeStruct((8, 128), jnp.float32),
)


def kernel(P, gumbel, Tau, mask_rows, mask_cols):
    p1 = jnp.pad(P.reshape(NPF), (0, NPF_PAD - NPF))
    g1 = jnp.pad(gumbel.reshape(NPF), (0, NPF_PAD - NPF))
    taub = jnp.full((16,), Tau, jnp.float32)
    rows_p = jnp.concatenate(
        [mask_rows.astype(jnp.int32), jnp.zeros((PADN,), jnp.int32)])
    cols_p = jnp.concatenate(
        [mask_cols.astype(jnp.int32), jnp.full((PADN,), NPF, jnp.int32)])
    d0, d1 = _sc_call(p1, g1, taub, rows_p, cols_p)
    acc = _red_call(d0.reshape(RED_ROWS, 128), d1.reshape(RED_ROWS, 128))
    return acc[0, 0]


# speedup vs baseline: 102.5515x; 1.0077x over previous
import jax
import jax.numpy as jnp
from jax import lax
from jax.experimental import pallas as pl
from jax.experimental.pallas import tpu as pltpu, tpu_sc as plsc

TWO_PIN_NET_NUM = 100000
L_PATTERN_NUM = 4
NPF = TWO_PIN_NET_NUM * L_PATTERN_NUM
G = 2 * 512 * 512
NNZ = 8000000
CAPACITY = 2.0
WCONG = 1.0

NC = 2
NS = 16
NW = NC * NS

PF_PER_SUB = 25008
NPF_PAD = NS * PF_PER_SUB
G_PER_SUB = G // NS
ZCH = 2048

ROWS = NNZ // 128
CHK = 16
ROWS_PAD = ((ROWS + 2 * NW * CHK - 1) // (2 * NW * CHK)) * (2 * NW * CHK)
RPW = ROWS_PAD // NW
N_CHUNKS = RPW // CHK
PADN = (ROWS_PAD - ROWS) * 128
PAD_LOCAL = NPF - (NS - 1) * PF_PER_SUB


_SHUF_DNUMS = lax.GatherDimensionNumbers(
    offset_dims=(), collapsed_slice_dims=(0,), start_index_map=(0,))


def _shuffle(x, idx2):
    return lax.gather(x, idx2, _SHUF_DNUMS, (1,),
                      mode=lax.GatherScatterMode.PROMISE_IN_BOUNDS)


def _sc_body(p1_hbm, g1_hbm, tau_hbm, rows_hbm, cols_hbm,
             d0_hbm, d1_hbm,
             pbuf, gbuf, taub, zbuf, colsb, rowsb, valsb,
             colsb2, rowsb2, valsb2,
             p_sh, dem_sh, ssem):
    c = lax.axis_index("c")
    s = lax.axis_index("s")
    wid = s * NC + c

    base_pf = s * PF_PER_SUB
    pltpu.sync_copy(p1_hbm.at[pl.ds(base_pf, PF_PER_SUB)], pbuf)
    pltpu.sync_copy(g1_hbm.at[pl.ds(base_pf, PF_PER_SUB)], gbuf)
    pltpu.sync_copy(tau_hbm, taub)
    tau = taub[...]
    lane = lax.iota(jnp.int32, 16)
    perm1 = (lane ^ 1)[:, None]
    perm2 = (lane ^ 2)[:, None]

    def sm_body(j, carry):
        b = j * 16
        x = (pbuf[pl.ds(b, 16)] + gbuf[pl.ds(b, 16)]) / tau
        m = jnp.maximum(x, _shuffle(x, perm1))
        m = jnp.maximum(m, _shuffle(m, perm2))
        e = jnp.exp(x - m)
        t = e + _shuffle(e, perm1)
        t = t + _shuffle(t, perm2)
        pbuf[pl.ds(b, 16)] = e / t
        return carry

    lax.fori_loop(0, PF_PER_SUB // 16, sm_body, 0)

    @pl.when(s == NS - 1)
    def _():
        def zp_body(k, carry):
            pbuf[pl.ds(PAD_LOCAL + k * 16, 16)] = jnp.zeros((16,), jnp.float32)
            return carry

        lax.fori_loop(0, (PF_PER_SUB - PAD_LOCAL) // 16, zp_body, 0)

    pltpu.sync_copy(pbuf, p_sh.at[pl.ds(base_pf, PF_PER_SUB)])

    def z_body(k, carry):
        zbuf[pl.ds(k * 16, 16)] = jnp.zeros((16,), jnp.float32)
        return carry

    lax.fori_loop(0, ZCH // 16, z_body, 0)

    def zc_body(k, carry):
        pltpu.sync_copy(zbuf, dem_sh.at[pl.ds(s * G_PER_SUB + k * ZCH, ZCH)])
        return carry

    lax.fori_loop(0, G_PER_SUB // ZCH, zc_body, 0)

    plsc.subcore_barrier()

    start = wid * RPW

    def c_body(t, carry):
        base0 = (start + 2 * t * CHK) * 128
        base1 = base0 + CHK * 128
        pltpu.sync_copy(cols_hbm.at[pl.ds(base0, CHK * 128)], colsb)
        pltpu.sync_copy(rows_hbm.at[pl.ds(base0, CHK * 128)], rowsb)
        pltpu.sync_copy(p_sh.at[colsb], valsb)
        sa = pltpu.async_copy(valsb, dem_sh.at[rowsb], ssem, add=True)
        pltpu.sync_copy(cols_hbm.at[pl.ds(base1, CHK * 128)], colsb2)
        pltpu.sync_copy(rows_hbm.at[pl.ds(base1, CHK * 128)], rowsb2)
        pltpu.sync_copy(p_sh.at[colsb2], valsb2)
        sa.wait()
        sb = pltpu.async_copy(valsb2, dem_sh.at[rowsb2], ssem, add=True)
        sb.wait()
        return carry

    lax.fori_loop(0, N_CHUNKS // 2, c_body, 0)

    plsc.subcore_barrier()

    off = s * G_PER_SUB

    @pl.when(c == 0)
    def _():
        pltpu.sync_copy(dem_sh.at[pl.ds(off, G_PER_SUB)],
                        d0_hbm.at[pl.ds(off, G_PER_SUB)])

    @pl.when(c == 1)
    def _():
        pltpu.sync_copy(dem_sh.at[pl.ds(off, G_PER_SUB)],
                        d1_hbm.at[pl.ds(off, G_PER_SUB)])


_sc_call = pl.kernel(
    _sc_body,
    out_type=(jax.ShapeDtypeStruct((G,), jnp.float32),
              jax.ShapeDtypeStruct((G,), jnp.float32)),
    mesh=plsc.VectorSubcoreMesh(core_axis_name="c", subcore_axis_name="s"),
    scratch_types=[
        pltpu.VMEM((PF_PER_SUB,), jnp.float32),
        pltpu.VMEM((PF_PER_SUB,), jnp.float32),
        pltpu.VMEM((16,), jnp.float32),
        pltpu.VMEM((ZCH,), jnp.float32),
        pltpu.VMEM((CHK * 128,), jnp.int32),
        pltpu.VMEM((CHK * 128,), jnp.int32),
        pltpu.VMEM((CHK * 128,), jnp.float32),
        pltpu.VMEM((CHK * 128,), jnp.int32),
        pltpu.VMEM((CHK * 128,), jnp.int32),
        pltpu.VMEM((CHK * 128,), jnp.float32),
        pltpu.VMEM_SHARED((NPF_PAD,), jnp.float32),
        pltpu.VMEM_SHARED((G,), jnp.float32),
        pltpu.SemaphoreType.DMA,
    ],
)

RED_ROWS = G // 128
RED_BLK = 512
RED_GRID = RED_ROWS // RED_BLK


def _red_body(d0_ref, d1_ref, o_ref):
    i = pl.program_id(0)
    x = d0_ref[...] + d1_ref[...]
    part = jnp.sum(jnp.maximum(x - CAPACITY, 0.0)) * WCONG

    @pl.when(i == 0)
    def _():
        o_ref[...] = jnp.zeros_like(o_ref)

    row = lax.broadcasted_iota(jnp.int32, (8, 128), 0)
    col = lax.broadcasted_iota(jnp.int32, (8, 128), 1)
    o_ref[...] += jnp.where((row == 0) & (col == 0), part, 0.0)


_red_call = pl.pallas_call(
    _red_body,
    grid=(RED_GRID,),
    in_specs=[pl.BlockSpec((RED_BLK, 128), lambda i: (i, 0)),
              pl.BlockSpec((RED_BLK, 128), lambda i: (i, 0))],
    out_specs=pl.BlockSpec((8, 128), lambda i: (0, 0)),
    out_shape=jax.ShapeDtypeStruct((8, 128), jnp.float32),
)


def kernel(P, gumbel, Tau, mask_rows, mask_cols):
    p1 = jnp.pad(P.reshape(NPF), (0, NPF_PAD - NPF))
    g1 = jnp.pad(gumbel.reshape(NPF), (0, NPF_PAD - NPF))
    taub = jnp.full((16,), Tau, jnp.float32)
    rows_p = jnp.concatenate(
        [mask_rows.astype(jnp.int32), jnp.zeros((PADN,), jnp.int32)])
    cols_p = jnp.concatenate(
        [mask_cols.astype(jnp.int32), jnp.full((PADN,), NPF, jnp.int32)])
    d0, d1 = _sc_call(p1, g1, taub, rows_p, cols_p)
    acc = _red_call(d0.reshape(RED_ROWS, 128), d1.reshape(RED_ROWS, 128))
    return acc[0, 0]
